# Initial kernel scaffold; baseline (speedup 1.0000x reference)
import jax
import jax.numpy as jnp
from jax.experimental import pallas as pl


def kernel(x, adj, W1, b1, W2, b2):
    raise NotImplementedError("write your pallas kernel here")


# speedup vs baseline: 1.0553x; 1.0553x over previous
import functools

import jax
import jax.numpy as jnp
from jax.experimental import pallas as pl
from jax.experimental.pallas import tpu as pltpu

N = 10000
F = 128
BM = 400


def _gcn_kernel(x_ref, adj_ref, w1_ref, b1_ref, w2_ref, b2_ref, out_ref,
                s1_ref, s2_ref):
    p = pl.program_id(0)
    i = pl.program_id(1)

    @pl.when(jnp.logical_and(p == 0, i == 0))
    def _():
        s1_ref[...] = jnp.dot(x_ref[...], w1_ref[...],
                              preferred_element_type=jnp.float32)

    @pl.when(p == 0)
    def _():
        h = jnp.dot(adj_ref[...], s1_ref[...],
                    preferred_element_type=jnp.float32)
        h = jnp.maximum(h + b1_ref[...], 0.0)
        s2_ref[pl.ds(i * BM, BM), :] = jnp.dot(
            h, w2_ref[...], preferred_element_type=jnp.float32)

    @pl.when(p == 1)
    def _():
        out_ref[...] = jnp.dot(adj_ref[...], s2_ref[...],
                               preferred_element_type=jnp.float32) + b2_ref[...]


@functools.partial(jax.jit, static_argnames=("interpret",))
def _gcn(x, adj, W1, b1, W2, b2, interpret=False):
    num_m = N // BM
    grid = (2, num_m)
    return pl.pallas_call(
        _gcn_kernel,
        grid=grid,
        in_specs=[
            pl.BlockSpec((N, F), lambda p, i: (0, 0)),
            pl.BlockSpec((BM, N), lambda p, i: (i, 0)),
            pl.BlockSpec((F, F), lambda p, i: (0, 0)),
            pl.BlockSpec((1, F), lambda p, i: (0, 0)),
            pl.BlockSpec((F, F), lambda p, i: (0, 0)),
            pl.BlockSpec((1, F), lambda p, i: (0, 0)),
        ],
        out_specs=pl.BlockSpec((BM, F), lambda p, i: (i, 0)),
        out_shape=jax.ShapeDtypeStruct((N, F), jnp.float32),
        scratch_shapes=[
            pltpu.VMEM((N, F), jnp.float32),
            pltpu.VMEM((N, F), jnp.float32),
        ],
        compiler_params=pltpu.CompilerParams(
            dimension_semantics=("arbitrary", "arbitrary"),
        ),
        interpret=interpret,
    )(x, adj, W1, b1, W2, b2)


def kernel(x, adj, W1, b1, W2, b2):
    return _gcn(x, adj, W1, b1.reshape(1, F), W2, b2.reshape(1, F))
